# Initial kernel scaffold; baseline (speedup 1.0000x reference)
import jax
import jax.numpy as jnp
from jax import lax
from jax.experimental import pallas as pl
from jax.experimental.pallas import tpu as pltpu
from jax.experimental.pallas import tpu_sc as plsc

NUM_EMB = 6
EMBED_DIM = 16
OUT_DIM = 32
N = 16384 * 26
NC = 2
NS = 16
NW = NC * NS
PER_W = N // NW
CHUNK = 1024
NCHUNKS = PER_W // CHUNK
GROUPS = CHUNK // 16
TBL = NUM_EMB * OUT_DIM


def _body(x_hbm, emb_hbm, w_hbm, b_hbm, out_hbm,
          idx_v, ob0, ob1, emb_v, w_v, b_v, tbl_v, sem0, sem1):
    cid = lax.axis_index("c")
    sid = lax.axis_index("s")
    wid = sid * NC + cid

    pltpu.sync_copy(emb_hbm, emb_v)
    pltpu.sync_copy(w_hbm, w_v)
    pltpu.sync_copy(b_hbm, b_v)

    for r in range(NUM_EMB):
        acc0 = b_v[pl.ds(0, 16)]
        acc1 = b_v[pl.ds(16, 16)]
        for k in range(EMBED_DIM):
            e = plsc.load_gather(
                emb_v, [jnp.full((16,), r * EMBED_DIM + k, jnp.int32)])
            acc0 = acc0 + e * w_v[pl.ds(k * OUT_DIM, 16)]
            acc1 = acc1 + e * w_v[pl.ds(k * OUT_DIM + 16, 16)]
        tbl_v[pl.ds(r * OUT_DIM, 16)] = acc0
        tbl_v[pl.ds(r * OUT_DIM + 16, 16)] = acc1

    iota = lax.iota(jnp.int32, 16)
    row_base = wid * PER_W

    obufs = (ob0, ob1)
    sems = (sem0, sem1)
    pending = [None, None]

    for ci in range(NCHUNKS):
        par = ci % 2
        ob = obufs[par]
        base = row_base + ci * CHUNK
        pltpu.sync_copy(x_hbm.at[pl.ds(base, CHUNK)], idx_v)
        if pending[par] is not None:
            pending[par].wait()

        def group(g, _, ob=ob):
            xt = idx_v[pl.ds(g * 16, 16)]
            gi = jnp.maximum(xt + 1, 0) * OUT_DIM
            si = iota * OUT_DIM + g * (16 * OUT_DIM)
            for c in range(OUT_DIM):
                gv = plsc.load_gather(tbl_v, [gi + c])
                plsc.store_scatter(ob, [si + c], gv)
            return 0

        lax.fori_loop(0, GROUPS, group, 0)

        pending[par] = pltpu.async_copy(
            ob, out_hbm.at[pl.ds(base * OUT_DIM, CHUNK * OUT_DIM)], sems[par])

    for p in pending:
        if p is not None:
            p.wait()


@jax.jit
def _run(x, embf, wf, b):
    f = pl.kernel(
        _body,
        out_type=jax.ShapeDtypeStruct((N * OUT_DIM,), jnp.float32),
        mesh=plsc.VectorSubcoreMesh(core_axis_name="c", subcore_axis_name="s"),
        scratch_types=[
            pltpu.VMEM((CHUNK,), jnp.int32),
            pltpu.VMEM((CHUNK * OUT_DIM,), jnp.float32),
            pltpu.VMEM((CHUNK * OUT_DIM,), jnp.float32),
            pltpu.VMEM((NUM_EMB * EMBED_DIM,), jnp.float32),
            pltpu.VMEM((EMBED_DIM * OUT_DIM,), jnp.float32),
            pltpu.VMEM((OUT_DIM,), jnp.float32),
            pltpu.VMEM((TBL,), jnp.float32),
            pltpu.SemaphoreType.DMA,
            pltpu.SemaphoreType.DMA,
        ],
    )
    return f(x, embf, wf, b)


def kernel(states, embed, W, b):
    x = states.reshape(-1).astype(jnp.int32)
    out = _run(x, embed.reshape(-1), W.reshape(-1), b)
    return out.reshape(N, OUT_DIM)


# speedup vs baseline: 2.0130x; 2.0130x over previous
import jax
import jax.numpy as jnp
from jax import lax
from jax.experimental import pallas as pl
from jax.experimental.pallas import tpu as pltpu
from jax.experimental.pallas import tpu_sc as plsc

NUM_EMB = 6
EMBED_DIM = 16
OUT_DIM = 32
N = 16384 * 26
NC = 2
NS = 16
NW = NC * NS
PER_W = N // NW
CHUNK = 1024
NCHUNKS = PER_W // CHUNK
GROUPS = CHUNK // 16
TBL = NUM_EMB * OUT_DIM


def _body(x_hbm, emb_hbm, w_hbm, b_hbm, out_hbm,
          idx_v, ob0, ob1, emb_v, w_v, b_v, tbl_v, sem0, sem1):
    cid = lax.axis_index("c")
    sid = lax.axis_index("s")
    wid = sid * NC + cid

    pltpu.sync_copy(emb_hbm, emb_v)
    pltpu.sync_copy(w_hbm, w_v)
    pltpu.sync_copy(b_hbm, b_v)

    for r in range(NUM_EMB):
        acc0 = b_v[pl.ds(0, 16)]
        acc1 = b_v[pl.ds(16, 16)]
        for k in range(EMBED_DIM):
            e = plsc.load_gather(
                emb_v, [jnp.full((16,), r * EMBED_DIM + k, jnp.int32)])
            acc0 = acc0 + e * w_v[pl.ds(k * OUT_DIM, 16)]
            acc1 = acc1 + e * w_v[pl.ds(k * OUT_DIM + 16, 16)]
        tbl_v[pl.ds(r * OUT_DIM, 16)] = acc0
        tbl_v[pl.ds(r * OUT_DIM + 16, 16)] = acc1

    iota = lax.iota(jnp.int32, 16)
    row_base = wid * PER_W

    obufs = (ob0, ob1)
    sems = (sem0, sem1)
    pending = [None, None]

    for ci in range(NCHUNKS):
        par = ci % 2
        ob = obufs[par]
        base = row_base + ci * CHUNK
        pltpu.sync_copy(x_hbm.at[pl.ds(base, CHUNK)], idx_v)
        if pending[par] is not None:
            pending[par].wait()

        def group(g, _, ob=ob):
            xt = idx_v[pl.ds(g * 16, 16)]
            gi = jnp.maximum(xt + 1, 0) * OUT_DIM
            si = iota * OUT_DIM + g * (16 * OUT_DIM)
            for c in range(OUT_DIM):
                gv = plsc.load_gather(tbl_v, [gi + c])
                plsc.store_scatter(ob, [si + c], gv)
            return 0

        lax.fori_loop(0, GROUPS, group, 0)

        pending[par] = pltpu.async_copy(
            ob, out_hbm.at[pl.ds(base * OUT_DIM, CHUNK * OUT_DIM)], sems[par])

    for p in pending:
        if p is not None:
            p.wait()


@jax.jit
def _run(x, embf, wf, b):
    f = pl.kernel(
        _body,
        out_type=jax.ShapeDtypeStruct((N * OUT_DIM,), jnp.float32),
        mesh=plsc.VectorSubcoreMesh(core_axis_name="c", subcore_axis_name="s"),
        scratch_types=[
            pltpu.VMEM((CHUNK,), jnp.int32),
            pltpu.VMEM((CHUNK * OUT_DIM,), jnp.float32),
            pltpu.VMEM((CHUNK * OUT_DIM,), jnp.float32),
            pltpu.VMEM((NUM_EMB * EMBED_DIM,), jnp.float32),
            pltpu.VMEM((EMBED_DIM * OUT_DIM,), jnp.float32),
            pltpu.VMEM((OUT_DIM,), jnp.float32),
            pltpu.VMEM((TBL,), jnp.float32),
            pltpu.SemaphoreType.DMA,
            pltpu.SemaphoreType.DMA,
        ],
        compiler_params=pltpu.CompilerParams(needs_layout_passes=False),
    )
    return f(x, embf, wf, b)


def kernel(states, embed, W, b):
    x = states.reshape(-1).astype(jnp.int32)
    out = _run(x, embed.reshape(-1), W.reshape(-1), b)
    return out.reshape(N, OUT_DIM)


# speedup vs baseline: 2.8639x; 1.4227x over previous
import jax
import jax.numpy as jnp
from jax import lax
from jax.experimental import pallas as pl
from jax.experimental.pallas import tpu as pltpu
from jax.experimental.pallas import tpu_sc as plsc

NUM_EMB = 6
EMBED_DIM = 16
OUT_DIM = 32
N = 16384 * 26
NC = 2
NS = 16
NW = NC * NS
PER_W = N // NW
CHUNK = 1024
NCHUNKS = PER_W // CHUNK
GROUPS = CHUNK // 16
TBL = NUM_EMB * OUT_DIM


def _body(x_hbm, emb_hbm, w_hbm, b_hbm, out_hbm,
          idx_v, ob0, ob1, emb_v, w_v, b_v, tbl_v, sem0, sem1):
    cid = lax.axis_index("c")
    sid = lax.axis_index("s")
    wid = sid * NC + cid

    pltpu.sync_copy(emb_hbm, emb_v)
    pltpu.sync_copy(w_hbm, w_v)
    pltpu.sync_copy(b_hbm, b_v)

    for r in range(NUM_EMB):
        acc0 = b_v[pl.ds(0, 16)]
        acc1 = b_v[pl.ds(16, 16)]
        for k in range(EMBED_DIM):
            e = plsc.load_gather(
                emb_v, [jnp.full((16,), r * EMBED_DIM + k, jnp.int32)])
            acc0 = acc0 + e * w_v[pl.ds(k * OUT_DIM, 16)]
            acc1 = acc1 + e * w_v[pl.ds(k * OUT_DIM + 16, 16)]
        tbl_v[pl.ds(r * OUT_DIM, 16)] = acc0
        tbl_v[pl.ds(r * OUT_DIM + 16, 16)] = acc1

    iota = lax.iota(jnp.int32, 16)
    row_base = wid * PER_W

    obufs = (ob0, ob1)
    sems = (sem0, sem1)
    pending = [None, None]

    for ci in range(NCHUNKS):
        par = ci % 2
        ob = obufs[par]
        base = row_base + ci * CHUNK
        pltpu.sync_copy(x_hbm.at[pl.ds(base, CHUNK)], idx_v)
        if pending[par] is not None:
            pending[par].wait()

        @plsc.parallel_loop(0, GROUPS)
        def group(g, ob=ob):
            xt = idx_v[pl.ds(g * 16, 16)]
            gi = jnp.maximum(xt + 1, 0) * OUT_DIM
            si = iota * OUT_DIM + g * (16 * OUT_DIM)
            for c in range(OUT_DIM):
                gv = plsc.load_gather(tbl_v, [gi + c])
                plsc.store_scatter(ob, [si + c], gv)

        pending[par] = pltpu.async_copy(
            ob, out_hbm.at[pl.ds(base * OUT_DIM, CHUNK * OUT_DIM)], sems[par])

    for p in pending:
        if p is not None:
            p.wait()


@jax.jit
def _run(x, embf, wf, b):
    f = pl.kernel(
        _body,
        out_type=jax.ShapeDtypeStruct((N * OUT_DIM,), jnp.float32),
        mesh=plsc.VectorSubcoreMesh(core_axis_name="c", subcore_axis_name="s"),
        scratch_types=[
            pltpu.VMEM((CHUNK,), jnp.int32),
            pltpu.VMEM((CHUNK * OUT_DIM,), jnp.float32),
            pltpu.VMEM((CHUNK * OUT_DIM,), jnp.float32),
            pltpu.VMEM((NUM_EMB * EMBED_DIM,), jnp.float32),
            pltpu.VMEM((EMBED_DIM * OUT_DIM,), jnp.float32),
            pltpu.VMEM((OUT_DIM,), jnp.float32),
            pltpu.VMEM((TBL,), jnp.float32),
            pltpu.SemaphoreType.DMA,
            pltpu.SemaphoreType.DMA,
        ],
        compiler_params=pltpu.CompilerParams(needs_layout_passes=False),
    )
    return f(x, embf, wf, b)


def kernel(states, embed, W, b):
    x = states.reshape(-1).astype(jnp.int32)
    out = _run(x, embed.reshape(-1), W.reshape(-1), b)
    return out.reshape(N, OUT_DIM)
